# baseline (speedup 1.0000x reference)
import functools

import jax
import jax.numpy as jnp
from jax import lax
from jax.experimental import pallas as pl
from jax.experimental.pallas import tpu as pltpu
from jax.experimental.pallas import tpu_sc as plsc

B = 4
S = 8192
D = 1024
NC = 2
NS = 16
NW = NC * NS
RPW = (B * S) // NW
WPB = S // RPW
C = 32
NCHUNK = RPW // C
NBUF = 2

_mesh = plsc.VectorSubcoreMesh(core_axis_name="c", subcore_axis_name="s")


@functools.partial(
    pl.kernel,
    mesh=_mesh,
    out_type=jax.ShapeDtypeStruct((B, S, D), jnp.float32),
    scratch_types=[
        pltpu.VMEM((RPW,), jnp.int32),
        pltpu.VMEM((NBUF * C, D), jnp.float32),
        pltpu.SemaphoreType.DMA,
        pltpu.SemaphoreType.DMA,
    ],
)
def _gather_rows(data_hbm, perm_hbm, out_hbm, idx_v, rows_v, gsem, ssem):
    wid = lax.axis_index("c") * NS + lax.axis_index("s")
    bi = wid // WPB
    lo = (wid % WPB) * RPW

    data_b = data_hbm.at[bi]
    out_b = out_hbm.at[bi]

    pltpu.sync_copy(perm_hbm.at[bi, pl.ds(lo, RPW)], idx_v)

    def _buf(b):
        return rows_v.at[pl.ds(b * C, C)]

    def _gdesc(c, b):
        return pltpu.make_async_copy(data_b.at[idx_v.at[pl.ds(c * C, C)]],
                                     _buf(b), gsem)

    def _sdesc(c, b):
        return pltpu.make_async_copy(_buf(b),
                                     out_b.at[pl.ds(lo + c * C, C)], ssem)

    def _step(c, carry):
        b = lax.rem(c, NBUF)

        @pl.when(c >= NBUF)
        def _wait_scatter():
            _sdesc(c - NBUF, b).wait()

        _gdesc(c, b).start()

        @pl.when(c >= 1)
        def _drain_prev():
            pb = lax.rem(c - 1, NBUF)
            _gdesc(c - 1, pb).wait()
            _sdesc(c - 1, pb).start()

        return carry

    lax.fori_loop(0, NCHUNK, _step, 0)

    last = NCHUNK - 1
    lb = last % NBUF
    _gdesc(last, lb).wait()
    _sdesc(last, lb).start()

    def _drain(i, carry):
        c = NCHUNK - NBUF + i
        _sdesc(c, lax.rem(c, NBUF)).wait()
        return carry

    lax.fori_loop(0, NBUF, _drain, 0)


def kernel(data, perm):
    return _gather_rows(data, perm)


# speedup vs baseline: 1.0038x; 1.0038x over previous
import functools

import jax
import jax.numpy as jnp
from jax import lax
from jax.experimental import pallas as pl
from jax.experimental.pallas import tpu as pltpu
from jax.experimental.pallas import tpu_sc as plsc

B = 4
S = 8192
D = 1024
NC = 2
NS = 16
NW = NC * NS
RPW = (B * S) // NW
WPB = S // RPW
C = 32
NCHUNK = RPW // C
NBUF = 3

_mesh = plsc.VectorSubcoreMesh(core_axis_name="c", subcore_axis_name="s")


@functools.partial(
    pl.kernel,
    mesh=_mesh,
    out_type=jax.ShapeDtypeStruct((B, S, D), jnp.float32),
    scratch_types=[
        pltpu.VMEM((RPW,), jnp.int32),
        pltpu.VMEM((NBUF * C, D), jnp.float32),
        pltpu.SemaphoreType.DMA,
        pltpu.SemaphoreType.DMA,
    ],
)
def _gather_rows(data_hbm, perm_hbm, out_hbm, idx_v, rows_v, gsem, ssem):
    wid = lax.axis_index("c") * NS + lax.axis_index("s")
    bi = wid // WPB
    lo = (wid % WPB) * RPW

    data_b = data_hbm.at[bi]
    out_b = out_hbm.at[bi]

    pltpu.sync_copy(perm_hbm.at[bi, pl.ds(lo, RPW)], idx_v)

    def _buf(b):
        return rows_v.at[pl.ds(b * C, C)]

    def _gdesc(c, b):
        return pltpu.make_async_copy(data_b.at[idx_v.at[pl.ds(c * C, C)]],
                                     _buf(b), gsem)

    def _sdesc(c, b):
        return pltpu.make_async_copy(_buf(b),
                                     out_b.at[pl.ds(lo + c * C, C)], ssem)

    def _step(c, carry):
        b = lax.rem(c, NBUF)

        @pl.when(c >= NBUF)
        def _wait_scatter():
            _sdesc(c - NBUF, b).wait()

        _gdesc(c, b).start()

        @pl.when(c >= 1)
        def _drain_prev():
            pb = lax.rem(c - 1, NBUF)
            _gdesc(c - 1, pb).wait()
            _sdesc(c - 1, pb).start()

        return carry

    lax.fori_loop(0, NCHUNK, _step, 0)

    last = NCHUNK - 1
    lb = last % NBUF
    _gdesc(last, lb).wait()
    _sdesc(last, lb).start()

    def _drain(i, carry):
        c = NCHUNK - NBUF + i
        _sdesc(c, lax.rem(c, NBUF)).wait()
        return carry

    lax.fori_loop(0, NBUF, _drain, 0)


def kernel(data, perm):
    return _gather_rows(data, perm)
